# Initial kernel scaffold; baseline (speedup 1.0000x reference)
import jax
import jax.numpy as jnp
from jax.experimental import pallas as pl


def kernel(x):
    raise NotImplementedError("write your pallas kernel here")


# speedup vs baseline: 19632.5497x; 19632.5497x over previous
import jax
import jax.numpy as jnp
from jax.experimental import pallas as pl
from jax.experimental.pallas import tpu as pltpu

_COLS = 4096
_BLK_ROWS = 512


def _absmax_body(x_ref, o_ref):
    @pl.when(pl.program_id(0) == 0)
    def _init():
        o_ref[0, 0] = jnp.float32(0.0)

    o_ref[0, 0] = jnp.maximum(o_ref[0, 0], jnp.max(jnp.abs(x_ref[...])))


def _quant_body(m_ref, x_ref, o_ref):
    x = x_ref[...]
    x_max = m_ref[0, 0]
    scale = x_max / jnp.float32(63.0)
    q = jnp.clip(jnp.round(x / scale), -64.0, 63.0)
    d = q * scale
    d = jnp.where(x_max > 0.0, d, x)
    o_ref[...] = x + (d - x)


def kernel(x):
    orig_shape = x.shape
    x2 = x.reshape(-1, _COLS)
    rows = x2.shape[0]
    grid = rows // _BLK_ROWS

    x_max = pl.pallas_call(
        _absmax_body,
        grid=(grid,),
        in_specs=[pl.BlockSpec((_BLK_ROWS, _COLS), lambda i: (i, 0))],
        out_specs=pl.BlockSpec(memory_space=pltpu.SMEM),
        out_shape=jax.ShapeDtypeStruct((1, 1), jnp.float32),
    )(x2)

    out = pl.pallas_call(
        _quant_body,
        grid=(grid,),
        in_specs=[
            pl.BlockSpec(memory_space=pltpu.SMEM),
            pl.BlockSpec((_BLK_ROWS, _COLS), lambda i: (i, 0)),
        ],
        out_specs=pl.BlockSpec((_BLK_ROWS, _COLS), lambda i: (i, 0)),
        out_shape=jax.ShapeDtypeStruct(x2.shape, jnp.float32),
        compiler_params=pltpu.CompilerParams(
            dimension_semantics=("parallel",),
        ),
    )(x_max, x2)

    return out.reshape(orig_shape)


# speedup vs baseline: 19869.5705x; 1.0121x over previous
import jax
import jax.numpy as jnp
from jax.experimental import pallas as pl
from jax.experimental.pallas import tpu as pltpu

_COLS = 4096
_BLK_ROWS = 512


def _fused_body(x_ref, o_ref, acc_ref):
    i = pl.program_id(0)
    half_steps = pl.num_programs(0) // 2

    @pl.when(i == 0)
    def _init():
        acc_ref[0, 0] = jnp.float32(0.0)

    @pl.when(i < half_steps)
    def _reduce():
        acc_ref[0, 0] = jnp.maximum(
            acc_ref[0, 0], jnp.max(jnp.abs(x_ref[...]))
        )

    @pl.when(i >= half_steps)
    def _quantize():
        x = x_ref[...]
        x_max = acc_ref[0, 0]
        scale = x_max / jnp.float32(63.0)
        q = jnp.clip(jnp.round(x / scale), -64.0, 63.0)
        d = q * scale
        d = jnp.where(x_max > 0.0, d, x)
        o_ref[...] = x + (d - x)


def kernel(x):
    orig_shape = x.shape
    x2 = x.reshape(-1, _COLS)
    rows = x2.shape[0]
    nblk = rows // _BLK_ROWS

    out = pl.pallas_call(
        _fused_body,
        grid=(2 * nblk,),
        in_specs=[pl.BlockSpec((_BLK_ROWS, _COLS), lambda i: (i % nblk, 0))],
        out_specs=pl.BlockSpec(
            (_BLK_ROWS, _COLS),
            lambda i: (jnp.maximum(i - nblk, 0), 0),
        ),
        out_shape=jax.ShapeDtypeStruct(x2.shape, jnp.float32),
        scratch_shapes=[pltpu.SMEM((1, 1), jnp.float32)],
    )(x2)

    return out.reshape(orig_shape)
